# baseline (speedup 1.0000x reference)
import jax
import jax.numpy as jnp
from jax.experimental import pallas as pl

_SEQ_BLK = 512


def _body(binT_ref, emb_ref, out_ref, logitT_ref):
    ampT = binT_ref[...] * 2.0 - 1.0
    table = emb_ref[...]
    out_ref[...] = ampT.T[:, :, None] * table[None, :, :]
    rowsum = jnp.sum(table, axis=1)
    logitT_ref[...] = ampT * rowsum[:, None]


def kernel(binary_input, embeddings):
    seq_len, blen = binary_input.shape
    vocab, emb_sz = embeddings.shape
    grid = (seq_len // _SEQ_BLK,)
    emb, logitT = pl.pallas_call(
        _body,
        grid=grid,
        in_specs=[
            pl.BlockSpec((blen, _SEQ_BLK), lambda i: (0, i)),
            pl.BlockSpec((vocab, emb_sz), lambda i: (0, 0)),
        ],
        out_specs=(
            pl.BlockSpec((_SEQ_BLK, blen, emb_sz), lambda i: (i, 0, 0)),
            pl.BlockSpec((blen, _SEQ_BLK), lambda i: (0, i)),
        ),
        out_shape=(
            jax.ShapeDtypeStruct((seq_len, blen, emb_sz), jnp.float32),
            jax.ShapeDtypeStruct((blen, seq_len), jnp.float32),
        ),
    )(binary_input.T, embeddings)
    return emb, logitT.T.reshape(seq_len, blen, 1)
